# baseline (speedup 1.0000x reference)
import functools

import jax
import jax.numpy as jnp
from jax import lax
from jax.experimental import pallas as pl
from jax.experimental.pallas import tpu as pltpu
from jax.experimental.pallas import tpu_sc as plsc

NC = 2
NS = 16
NW = NC * NS


def _transpose_body(w_ref, out_ref):
    out_ref[...] = w_ref[...].T


def _make_sc_gather(B, OBS, ACT):
    BPW = B // NW
    CH = 32
    NCH = BPW // CH

    mesh = plsc.VectorSubcoreMesh(
        core_axis_name="c", subcore_axis_name="s",
        num_cores=NC, num_subcores=NS)

    @functools.partial(
        pl.kernel,
        mesh=mesh,
        compiler_params=pltpu.CompilerParams(use_tc_tiling_on_sc=False),
        out_type=(
            jax.ShapeDtypeStruct((B, ACT), jnp.float32),
            jax.ShapeDtypeStruct((B,), jnp.float32),
            jax.ShapeDtypeStruct((B, ACT), jnp.float32),
        ),
        scratch_types=[
            pltpu.VMEM((BPW,), jnp.int32),
            pltpu.VMEM((CH, ACT), jnp.float32),
            pltpu.VMEM((CH, ACT), jnp.float32),
            pltpu.VMEM((BPW,), jnp.float32),
            pltpu.SemaphoreType.DMA,
            pltpu.SemaphoreType.DMA,
        ],
    )
    def sc_gather(x_hbm, v_hbm, q_hbm, piT_hbm,
                  pi_hbm, val_hbm, qval_hbm,
                  idx_v, rows_q, rows_p, val_v, sem_q, sem_p):
        wid = lax.axis_index("s") * NC + lax.axis_index("c")
        base = wid * BPW
        pltpu.sync_copy(x_hbm.at[pl.ds(base, BPW)], idx_v)
        pltpu.async_copy(v_hbm.at[idx_v], val_v, sem_q).wait()
        pltpu.sync_copy(val_v, val_hbm.at[pl.ds(base, BPW)])

        def chunk(i, carry):
            off = i * CH
            row0 = base + off
            idx_c = idx_v.at[pl.ds(off, CH)]
            cq = pltpu.async_copy(q_hbm.at[idx_c], rows_q, sem_q)
            cp = pltpu.async_copy(piT_hbm.at[idx_c], rows_p, sem_p)
            cq.wait()
            pltpu.sync_copy(rows_q, qval_hbm.at[pl.ds(row0, CH)])
            cp.wait()
            pltpu.sync_copy(rows_p, pi_hbm.at[pl.ds(row0, CH)])
            return carry

        lax.fori_loop(0, NCH, chunk, 0)

    return sc_gather


def kernel(x, v, q, pi_w):
    B = x.shape[0]
    ACT, OBS = pi_w.shape
    x32 = x.astype(jnp.int32)
    piT = pl.pallas_call(
        _transpose_body,
        out_shape=jax.ShapeDtypeStruct((OBS, ACT), jnp.float32),
    )(pi_w)
    pi_out, val, qval = _make_sc_gather(B, OBS, ACT)(x32, v, q, piT)
    return (pi_out, val, qval)


# speedup vs baseline: 1.2131x; 1.2131x over previous
import functools

import jax
import jax.numpy as jnp
from jax import lax
from jax.experimental import pallas as pl
from jax.experimental.pallas import tpu as pltpu
from jax.experimental.pallas import tpu_sc as plsc

NC = 2
NS = 16
NW = NC * NS


def _make_sc(B, OBS, ACT):
    BPW = B // NW
    CH = 128
    NCH = BPW // CH
    JB = (ACT + 127) // 128
    JW_LAST = ACT - 128 * (JB - 1)
    NT = 2
    ROWS = NT * JB
    NOPS = NCH * ROWS

    mesh = plsc.VectorSubcoreMesh(
        core_axis_name="c", subcore_axis_name="s",
        num_cores=NC, num_subcores=NS)

    @functools.partial(
        pl.kernel,
        mesh=mesh,
        compiler_params=pltpu.CompilerParams(use_tc_tiling_on_sc=True),
        out_type=(
            jax.ShapeDtypeStruct((B, ACT), jnp.float32),
            jax.ShapeDtypeStruct((B,), jnp.float32),
            jax.ShapeDtypeStruct((B, ACT), jnp.float32),
            jax.ShapeDtypeStruct((B, 2 * 128), jnp.float32),
        ),
        scratch_types=[
            pltpu.VMEM((BPW,), jnp.int32),
            pltpu.VMEM((NOPS, CH), jnp.int32),
            pltpu.VMEM((CH, 128), jnp.float32),
            pltpu.VMEM((CH, 128), jnp.float32),
            pltpu.VMEM((BPW,), jnp.float32),
            pltpu.SemaphoreType.DMA,
            pltpu.SemaphoreType.DMA,
            pltpu.SemaphoreType.DMA,
            pltpu.SemaphoreType.DMA,
        ],
    )
    def sc_gather(x_hbm, v_hbm, tab_hbm,
                  pi_hbm, val_hbm, qval_hbm, st_hbm,
                  idx_v, idxj_v, S0, S1, val_v, sg0, sg1, sw0, sw1):
        wid = lax.axis_index("s") * NC + lax.axis_index("c")
        base = wid * BPW
        pltpu.sync_copy(x_hbm.at[pl.ds(base, BPW)], idx_v)

        pltpu.async_copy(v_hbm.at[idx_v], val_v, sg0).wait()
        pltpu.sync_copy(val_v, val_hbm.at[pl.ds(base, BPW)])

        for c in range(NCH):
            for u in range(CH // 16):
                vec = idx_v[pl.ds(c * CH + u * 16, 16)]
                for r in range(ROWS):
                    idxj_v[c * ROWS + r, pl.ds(u * 16, 16)] = vec + r * OBS

        S = (S0, S1)
        sg = (sg0, sg1)
        sw = (sw0, sw1)
        ops = [(c, t, j) for c in range(NCH) for t in range(NT)
               for j in range(JB)]

        def fire_gather(n):
            c, t, j = ops[n]
            b = n % 2
            return pltpu.async_copy(
                tab_hbm.at[idxj_v.at[c * ROWS + t * JB + j]], S[b], sg[b])

        def fire_write(n):
            c, t, j = ops[n]
            b = n % 2
            if j < JB - 1:
                out = qval_hbm if t == 0 else pi_hbm
                dst = out.at[pl.ds(base + c * CH, CH), pl.ds(j * 128, 128)]
            else:
                dst = st_hbm.at[pl.ds(base + c * CH, CH),
                                pl.ds(t * 128, 128)]
            return pltpu.async_copy(S[b], dst, sw[b])

        gd = {0: fire_gather(0), 1: fire_gather(1)}
        wd = {}
        for n in range(NOPS):
            gd[n].wait()
            wd[n] = fire_write(n)
            if n + 2 < NOPS:
                wd[n].wait()
                gd[n + 2] = fire_gather(n + 2)
        wd[NOPS - 2].wait()
        wd[NOPS - 1].wait()

    return sc_gather


def _finish_body(st_ref, q_in, p_in, q_ref, p_ref):
    del q_in, p_in
    q_ref[...] = st_ref[:, 0:128]
    p_ref[...] = st_ref[:, 128:256]


def _finish(st, qval, pi_out, B, ACT):
    JB = (ACT + 127) // 128
    BR = 1024
    return pl.pallas_call(
        _finish_body,
        grid=(B // BR,),
        in_specs=[
            pl.BlockSpec((BR, 256), lambda i: (i, 0)),
            pl.BlockSpec(memory_space=pl.ANY),
            pl.BlockSpec(memory_space=pl.ANY),
        ],
        out_specs=[
            pl.BlockSpec((BR, 128), lambda i: (i, JB - 1)),
            pl.BlockSpec((BR, 128), lambda i: (i, JB - 1)),
        ],
        out_shape=[
            jax.ShapeDtypeStruct((B, ACT), jnp.float32),
            jax.ShapeDtypeStruct((B, ACT), jnp.float32),
        ],
        input_output_aliases={1: 0, 2: 1},
    )(st, qval, pi_out)


def kernel(x, v, q, pi_w):
    B = x.shape[0]
    ACT, OBS = pi_w.shape
    JB = (ACT + 127) // 128
    PAD = JB * 128 - ACT
    x32 = x.astype(jnp.int32)
    qp = jnp.pad(q, ((0, 0), (0, PAD)))
    q3 = qp.reshape(OBS, JB, 128).transpose(1, 0, 2)
    pp = jnp.pad(pi_w, ((0, PAD), (0, 0)))
    p3 = pp.reshape(JB, 128, OBS).transpose(0, 2, 1)
    tab = jnp.concatenate([q3, p3], axis=0).reshape(2 * JB * OBS, 128)
    pi_sc, val, qval_sc, st = _make_sc(B, OBS, ACT)(x32, v, tab)
    qval, pi_out = _finish(st, qval_sc, pi_sc, B, ACT)
    return (pi_out, val, qval)
